# Initial kernel scaffold; baseline (speedup 1.0000x reference)
import jax
import jax.numpy as jnp
from jax.experimental import pallas as pl


def kernel(x, edge_index, params):
    raise NotImplementedError("write your pallas kernel here")


# speedup vs baseline: 20.2880x; 20.2880x over previous
import functools

import jax
import jax.numpy as jnp
from jax import lax
from jax.experimental import pallas as pl
from jax.experimental.pallas import tpu as pltpu
from jax.experimental.pallas import tpu_sc as plsc

HEADS = 2
H = 128
LANES = 16
NT = 16
CHUNK = 256
K = CHUNK // 128


def _dot_bf16(x, y):
    return jnp.dot(x.astype(jnp.bfloat16), y.astype(jnp.bfloat16),
                   preferred_element_type=jnp.float32)


def _tc_prep_body(h_ref, w_ref, asrc_ref, adst_ref, z_ref, s_ref, d_ref):
    h = h_ref[...]
    for i in range(HEADS):
        z = _dot_bf16(h, w_ref[i])
        z_ref[i] = z
        s_ref[i] = _dot_bf16(z, asrc_ref[i])
        d_ref[i] = _dot_bf16(z, adst_ref[i])


def _tc_combine_body(zacc_ref, den_ref, gamma_ref, beta_ref, w_ref, b_ref,
                     hnext_ref, hmean_ref, *, n):
    outs = []
    for i in range(HEADS):
        den = den_ref[i, :n][:, None]
        den = jnp.where(den == 0.0, 1.0, den)
        hh = jnp.maximum(zacc_ref[i, :n] / den, 0.0)
        mu = jnp.mean(hh, axis=0)
        var = jnp.mean((hh - mu) ** 2, axis=0)
        hh = (hh - mu) * lax.rsqrt(var + 1e-5) * gamma_ref[i] + beta_ref[i]
        outs.append(hh)
    w = w_ref[...]
    t0 = _dot_bf16(outs[0], w) + b_ref[...]
    t1 = _dot_bf16(outs[1], w) + b_ref[...]
    m = jnp.maximum(t0, t1)
    e0 = jnp.exp(t0 - m)
    e1 = jnp.exp(t1 - m)
    inv = 1.0 / (e0 + e1)
    rb = lambda x: x.astype(jnp.bfloat16).astype(jnp.float32)
    hn = rb(e0 * inv) * rb(outs[0]) + rb(e1 * inv) * rb(outs[1])
    hnext_ref[...] = hn
    hmean_ref[...] = jnp.mean(hn, axis=0)


@functools.cache
def _make_edge_kernel(n, e):
    npad = ((n + 8 * NT - 1) // (8 * NT)) * (8 * NT)
    nchunk = e // CHUNK
    rows_per_tile = npad // NT
    outer = (nchunk + NT - 1) // NT
    mesh = plsc.VectorSubcoreMesh(core_axis_name="c", subcore_axis_name="s")

    @functools.partial(
        pl.kernel,
        out_type=(jax.ShapeDtypeStruct((HEADS, npad, H), jnp.float32),
                  jax.ShapeDtypeStruct((HEADS * npad,), jnp.float32)),
        mesh=mesh,
        scratch_types=[
            pltpu.VMEM((K, 128), jnp.int32),
            pltpu.VMEM((K, 128), jnp.int32),
            pltpu.VMEM((K, 128), jnp.int32),
            pltpu.VMEM((CHUNK, H), jnp.float32),
            pltpu.VMEM((CHUNK,), jnp.float32),
            pltpu.VMEM((CHUNK,), jnp.float32),
            pltpu.VMEM((CHUNK,), jnp.float32),
            pltpu.VMEM((rows_per_tile,), jnp.float32),
            pltpu.VMEM_SHARED((npad, H), jnp.float32),
            pltpu.VMEM_SHARED((npad,), jnp.float32),
            pltpu.SemaphoreType.DMA,
            pltpu.SemaphoreType.DMA,
        ],
        compiler_params=pltpu.CompilerParams(needs_layout_passes=False),
    )
    def edge_kernel(src_hbm, dst_hbm, z_hbm, s_hbm, d_hbm,
                    zout_hbm, dout_hbm,
                    zidx, didx, widx, zbuf, sbuf, dbuf, aebuf, dbounce,
                    zacc, dacc, sem, sem2):
        c = lax.axis_index("c")
        t = lax.axis_index("s")
        cn = c * n

        zero = jnp.zeros((LANES,), jnp.float32)

        def zrow(i, carry):
            for r in range(H // LANES):
                zbuf[i, pl.ds(r * LANES, LANES)] = zero
            return carry

        lax.fori_loop(0, CHUNK, zrow, 0)
        for g in range(CHUNK // LANES):
            aebuf[pl.ds(g * LANES, LANES)] = zero
        row0 = t * rows_per_tile
        done = 0
        while done < rows_per_tile:
            nrows = min(CHUNK, rows_per_tile - done)
            pltpu.sync_copy(zbuf.at[pl.ds(0, nrows)],
                            zacc.at[pl.ds(row0 + done, nrows)])
            pltpu.sync_copy(aebuf.at[pl.ds(0, nrows)],
                            dacc.at[pl.ds(row0 + done, nrows)])
            done += nrows
        plsc.subcore_barrier()

        def chunk_body(kk, carry):
            chunk_id = t + NT * kk

            @pl.when(chunk_id < nchunk)
            def _():
                base_row = chunk_id * K
                pltpu.sync_copy(src_hbm.at[pl.ds(base_row, K)], zidx)
                pltpu.sync_copy(dst_hbm.at[pl.ds(base_row, K)], didx)
                for j in range(K):
                    for g in range(128 // LANES):
                        sl = pl.ds(g * LANES, LANES)
                        zidx[j, sl] = zidx[j, sl] + cn
                        widx[j, sl] = didx[j, sl] + cn
                cps = [pltpu.async_copy(z_hbm.at[zidx.at[j]],
                                        zbuf.at[pl.ds(j * 128, 128)], sem)
                       for j in range(K)]
                sd = []
                for j in range(K):
                    sl = pl.ds(j * 128, 128)
                    sd.append(pltpu.async_copy(s_hbm.at[zidx.at[j]],
                                               sbuf.at[sl], sem2))
                    sd.append(pltpu.async_copy(d_hbm.at[widx.at[j]],
                                               dbuf.at[sl], sem2))
                for cp in sd:
                    cp.wait()
                for g in range(CHUNK // LANES):
                    sl = pl.ds(g * LANES, LANES)
                    ev = sbuf[sl] + dbuf[sl]
                    ev = jnp.maximum(ev, ev * 0.01)
                    aebuf[sl] = jnp.exp(ev)
                for cp in cps:
                    cp.wait()

                def scale_body(i, carry):
                    bidx = jnp.zeros((LANES,), jnp.int32) + i
                    w = plsc.load_gather(aebuf, [bidx])
                    for r in range(H // LANES):
                        sl = pl.ds(r * LANES, LANES)
                        zbuf[i, sl] = zbuf[i, sl] * w
                    return carry

                lax.fori_loop(0, CHUNK, scale_body, 0, unroll=2)

                for j in range(K):
                    pltpu.sync_copy(zbuf.at[pl.ds(j * 128, 128)],
                                    zacc.at[didx.at[j]], add=True)
                    pltpu.sync_copy(aebuf.at[pl.ds(j * 128, 128)],
                                    dacc.at[didx.at[j]], add=True)

            return carry

        lax.fori_loop(0, outer, chunk_body, 0)
        plsc.subcore_barrier()

        pltpu.sync_copy(zacc.at[pl.ds(row0, rows_per_tile)],
                        zout_hbm.at[c, pl.ds(row0, rows_per_tile)])
        pltpu.sync_copy(dacc.at[pl.ds(row0, rows_per_tile)], dbounce)
        pltpu.sync_copy(dbounce,
                        dout_hbm.at[pl.ds(c * npad + row0, rows_per_tile)])

    return edge_kernel


def kernel(x, edge_index, params):
    n, d_in = x.shape
    e = edge_index.shape[1]
    src = edge_index[0].astype(jnp.int32).reshape(e // 128, 128)
    dst = edge_index[1].astype(jnp.int32).reshape(e // 128, 128)
    edge_fn = _make_edge_kernel(n, e)
    npad = ((n + 8 * NT - 1) // (8 * NT)) * (8 * NT)

    tc_prep = pl.pallas_call(
        _tc_prep_body,
        out_shape=(jax.ShapeDtypeStruct((HEADS, n, H), jnp.float32),
                   jax.ShapeDtypeStruct((HEADS, n), jnp.float32),
                   jax.ShapeDtypeStruct((HEADS, n), jnp.float32)),
    )
    tc_combine = pl.pallas_call(
        functools.partial(_tc_combine_body, n=n),
        out_shape=(jax.ShapeDtypeStruct((n, H), jnp.float32),
                   jax.ShapeDtypeStruct((H,), jnp.float32)),
    )

    h = x
    hmean = None
    for j in range(2):
        w_stack = jnp.stack([params['conv{}_{}'.format(j, i)]['W']
                             for i in range(HEADS)])
        asrc = jnp.stack([params['conv{}_{}'.format(j, i)]['a_src']
                          for i in range(HEADS)])
        adst = jnp.stack([params['conv{}_{}'.format(j, i)]['a_dst']
                          for i in range(HEADS)])
        gamma = jnp.stack([params['conv{}_{}'.format(j, i)]['gamma']
                           for i in range(HEADS)])
        beta = jnp.stack([params['conv{}_{}'.format(j, i)]['beta']
                          for i in range(HEADS)])
        z, s, d = tc_prep(h, w_stack, asrc, adst)
        zacc, den = edge_fn(src, dst, z.reshape(HEADS * n, H),
                            s.reshape(HEADS * n), d.reshape(HEADS * n))
        den = den.reshape(HEADS, npad)
        h, hmean = tc_combine(zacc, den, gamma, beta,
                              params['out{}'.format(j)]['w'],
                              params['out{}'.format(j)]['b'])
    return hmean


# speedup vs baseline: 27.1636x; 1.3389x over previous
import functools

import jax
import jax.numpy as jnp
from jax import lax
from jax.experimental import pallas as pl
from jax.experimental.pallas import tpu as pltpu
from jax.experimental.pallas import tpu_sc as plsc

HEADS = 2
H = 128
LANES = 16
NT = 16
CHUNK = 128


def _dot_bf16(x, y):
    return jnp.dot(x.astype(jnp.bfloat16), y.astype(jnp.bfloat16),
                   preferred_element_type=jnp.float32)


def _tc_prep_body(h_ref, w_ref, asrc_ref, adst_ref, z_ref, s_ref, d_ref):
    h = h_ref[...]
    for i in range(HEADS):
        z = _dot_bf16(h, w_ref[i])
        z_ref[i] = z
        s_ref[i] = _dot_bf16(z, asrc_ref[i])
        d_ref[i] = _dot_bf16(z, adst_ref[i])


def _tc_combine_body(zacc_ref, den_ref, gamma_ref, beta_ref, w_ref, b_ref,
                     hnext_ref, hmean_ref, *, n):
    outs = []
    for i in range(HEADS):
        den = den_ref[i, :n][:, None]
        den = jnp.where(den == 0.0, 1.0, den)
        hh = jnp.maximum(zacc_ref[i, :n] / den, 0.0)
        mu = jnp.mean(hh, axis=0)
        var = jnp.mean((hh - mu) ** 2, axis=0)
        hh = (hh - mu) * lax.rsqrt(var + 1e-5) * gamma_ref[i] + beta_ref[i]
        outs.append(hh)
    w = w_ref[...]
    t0 = _dot_bf16(outs[0], w) + b_ref[...]
    t1 = _dot_bf16(outs[1], w) + b_ref[...]
    m = jnp.maximum(t0, t1)
    e0 = jnp.exp(t0 - m)
    e1 = jnp.exp(t1 - m)
    inv = 1.0 / (e0 + e1)
    rb = lambda x: x.astype(jnp.bfloat16).astype(jnp.float32)
    hn = rb(e0 * inv) * rb(outs[0]) + rb(e1 * inv) * rb(outs[1])
    hnext_ref[...] = hn
    hmean_ref[...] = jnp.mean(hn, axis=0)


@functools.cache
def _make_edge_kernel(n, e):
    npad = ((n + 8 * NT - 1) // (8 * NT)) * (8 * NT)
    nchunk = e // CHUNK
    rows_per_tile = npad // NT
    outer = (nchunk + NT - 1) // NT
    mesh = plsc.VectorSubcoreMesh(core_axis_name="c", subcore_axis_name="s")

    @functools.partial(
        pl.kernel,
        out_type=(jax.ShapeDtypeStruct((HEADS, npad, H), jnp.float32),
                  jax.ShapeDtypeStruct((HEADS * npad,), jnp.float32)),
        mesh=mesh,
        scratch_types=[
            pltpu.VMEM((2, 1, 128), jnp.int32),
            pltpu.VMEM((2, 1, 128), jnp.int32),
            pltpu.VMEM((2, 1, 128), jnp.int32),
            pltpu.VMEM((2, CHUNK, H), jnp.float32),
            pltpu.VMEM((2, CHUNK), jnp.float32),
            pltpu.VMEM((2, CHUNK), jnp.float32),
            pltpu.VMEM((2, CHUNK), jnp.float32),
            pltpu.VMEM((rows_per_tile,), jnp.float32),
            pltpu.VMEM_SHARED((npad, H), jnp.float32),
            pltpu.VMEM_SHARED((npad,), jnp.float32),
            pltpu.SemaphoreType.DMA,
            pltpu.SemaphoreType.DMA,
            pltpu.SemaphoreType.DMA,
            pltpu.SemaphoreType.DMA,
            pltpu.SemaphoreType.DMA,
            pltpu.SemaphoreType.DMA,
        ],
        compiler_params=pltpu.CompilerParams(needs_layout_passes=False),
    )
    def edge_kernel(src_hbm, dst_hbm, z_hbm, s_hbm, d_hbm,
                    zout_hbm, dout_hbm,
                    zidx, didx, widx, zbuf, sbuf, dbuf, aebuf, dbounce,
                    zacc, dacc, semi0, semi1, semz0, semz1, semsd0, semsd1):
        c = lax.axis_index("c")
        t = lax.axis_index("s")
        cn = c * n
        semi = (semi0, semi1)
        semz = (semz0, semz1)
        semsd = (semsd0, semsd1)

        zero = jnp.zeros((LANES,), jnp.float32)

        def zrow(i, carry):
            for r in range(H // LANES):
                zbuf[0, i, pl.ds(r * LANES, LANES)] = zero
            return carry

        lax.fori_loop(0, CHUNK, zrow, 0)
        for g in range(CHUNK // LANES):
            aebuf[0, pl.ds(g * LANES, LANES)] = zero
        row0 = t * rows_per_tile
        done = 0
        while done < rows_per_tile:
            nrows = min(CHUNK, rows_per_tile - done)
            pltpu.sync_copy(zbuf.at[0, pl.ds(0, nrows)],
                            zacc.at[pl.ds(row0 + done, nrows)])
            pltpu.sync_copy(aebuf.at[0, pl.ds(0, nrows)],
                            dacc.at[pl.ds(row0 + done, nrows)])
            done += nrows
        plsc.subcore_barrier()

        def cid_of(kk):
            return t + NT * kk

        def fire_idx(b, cid):
            @pl.when(cid < nchunk)
            def _():
                pltpu.async_copy(src_hbm.at[pl.ds(cid, 1)], zidx.at[b],
                                 semi[b])
                pltpu.async_copy(dst_hbm.at[pl.ds(cid, 1)], didx.at[b],
                                 semi[b])

        def fire_gathers(b, cid):
            @pl.when(cid < nchunk)
            def _():
                pltpu.make_async_copy(src_hbm.at[pl.ds(cid, 1)], zidx.at[b],
                                      semi[b]).wait()
                pltpu.make_async_copy(dst_hbm.at[pl.ds(cid, 1)], didx.at[b],
                                      semi[b]).wait()
                for g in range(128 // LANES):
                    sl = pl.ds(g * LANES, LANES)
                    zidx[b, 0, sl] = zidx[b, 0, sl] + cn
                    widx[b, 0, sl] = didx[b, 0, sl] + cn
                pltpu.async_copy(z_hbm.at[zidx.at[b, 0]], zbuf.at[b], semz[b])
                pltpu.async_copy(s_hbm.at[zidx.at[b, 0]], sbuf.at[b],
                                 semsd[b])
                pltpu.async_copy(d_hbm.at[widx.at[b, 0]], dbuf.at[b],
                                 semsd[b])

        def work(b, cid):
            @pl.when(cid < nchunk)
            def _():
                pltpu.make_async_copy(s_hbm.at[zidx.at[b, 0]], sbuf.at[b],
                                      semsd[b]).wait()
                pltpu.make_async_copy(d_hbm.at[widx.at[b, 0]], dbuf.at[b],
                                      semsd[b]).wait()
                for g in range(CHUNK // LANES):
                    sl = pl.ds(g * LANES, LANES)
                    ev = sbuf[b, sl] + dbuf[b, sl]
                    ev = jnp.maximum(ev, ev * 0.01)
                    aebuf[b, sl] = jnp.exp(ev)
                pltpu.make_async_copy(z_hbm.at[zidx.at[b, 0]], zbuf.at[b],
                                      semz[b]).wait()

                def scale_body(i, carry):
                    bidx = jnp.zeros((LANES,), jnp.int32) + i
                    w = plsc.load_gather(aebuf.at[b], [bidx])
                    for r in range(H // LANES):
                        sl = pl.ds(r * LANES, LANES)
                        zbuf[b, i, sl] = zbuf[b, i, sl] * w
                    return carry

                lax.fori_loop(0, CHUNK, scale_body, 0, unroll=2)
                pltpu.sync_copy(zbuf.at[b], zacc.at[didx.at[b, 0]], add=True)
                pltpu.sync_copy(aebuf.at[b], dacc.at[didx.at[b, 0]],
                                add=True)

        fire_idx(0, cid_of(0))
        fire_idx(1, cid_of(1))
        fire_gathers(0, cid_of(0))

        def pair_body(kk2, carry):
            kk = 2 * kk2
            for b in (0, 1):
                fire_gathers(1 - b, cid_of(kk + 1))
                work(b, cid_of(kk))
                fire_idx(b, cid_of(kk + 2))
                kk = kk + 1
            return carry

        lax.fori_loop(0, (outer + 1) // 2, pair_body, 0)
        plsc.subcore_barrier()

        pltpu.sync_copy(zacc.at[pl.ds(row0, rows_per_tile)],
                        zout_hbm.at[c, pl.ds(row0, rows_per_tile)])
        pltpu.sync_copy(dacc.at[pl.ds(row0, rows_per_tile)], dbounce)
        pltpu.sync_copy(dbounce,
                        dout_hbm.at[pl.ds(c * npad + row0, rows_per_tile)])

    return edge_kernel


def kernel(x, edge_index, params):
    n, d_in = x.shape
    e = edge_index.shape[1]
    src = edge_index[0].astype(jnp.int32).reshape(e // 128, 128)
    dst = edge_index[1].astype(jnp.int32).reshape(e // 128, 128)
    edge_fn = _make_edge_kernel(n, e)
    npad = ((n + 8 * NT - 1) // (8 * NT)) * (8 * NT)

    tc_prep = pl.pallas_call(
        _tc_prep_body,
        out_shape=(jax.ShapeDtypeStruct((HEADS, n, H), jnp.float32),
                   jax.ShapeDtypeStruct((HEADS, n), jnp.float32),
                   jax.ShapeDtypeStruct((HEADS, n), jnp.float32)),
    )
    tc_combine = pl.pallas_call(
        functools.partial(_tc_combine_body, n=n),
        out_shape=(jax.ShapeDtypeStruct((n, H), jnp.float32),
                   jax.ShapeDtypeStruct((H,), jnp.float32)),
    )

    h = x
    hmean = None
    for j in range(2):
        w_stack = jnp.stack([params['conv{}_{}'.format(j, i)]['W']
                             for i in range(HEADS)])
        asrc = jnp.stack([params['conv{}_{}'.format(j, i)]['a_src']
                          for i in range(HEADS)])
        adst = jnp.stack([params['conv{}_{}'.format(j, i)]['a_dst']
                          for i in range(HEADS)])
        gamma = jnp.stack([params['conv{}_{}'.format(j, i)]['gamma']
                           for i in range(HEADS)])
        beta = jnp.stack([params['conv{}_{}'.format(j, i)]['beta']
                          for i in range(HEADS)])
        z, s, d = tc_prep(h, w_stack, asrc, adst)
        zacc, den = edge_fn(src, dst, z.reshape(HEADS * n, H),
                            s.reshape(HEADS * n), d.reshape(HEADS * n))
        den = den.reshape(HEADS, npad)
        h, hmean = tc_combine(zacc, den, gamma, beta,
                              params['out{}'.format(j)]['w'],
                              params['out{}'.format(j)]['b'])
    return hmean


# speedup vs baseline: 27.3346x; 1.0063x over previous
import functools

import jax
import jax.numpy as jnp
from jax import lax
from jax.experimental import pallas as pl
from jax.experimental.pallas import tpu as pltpu
from jax.experimental.pallas import tpu_sc as plsc

HEADS = 2
H = 128
LANES = 16
NT = 16
CHUNK = 128


def _dot_bf16(x, y):
    return jnp.dot(x.astype(jnp.bfloat16), y.astype(jnp.bfloat16),
                   preferred_element_type=jnp.float32)


def _tc_prep_body(h_ref, w_ref, asrc_ref, adst_ref, z_ref, s_ref, d_ref):
    h = h_ref[...]
    for i in range(HEADS):
        z = _dot_bf16(h, w_ref[i])
        z_ref[i] = z
        s_ref[i] = _dot_bf16(z, asrc_ref[i])
        d_ref[i] = _dot_bf16(z, adst_ref[i])


def _tc_combine_body(zacc_ref, den_ref, gamma_ref, beta_ref, w_ref, b_ref,
                     hnext_ref, hmean_ref, *, n):
    outs = []
    for i in range(HEADS):
        den = den_ref[i, :n][:, None]
        den = jnp.where(den == 0.0, 1.0, den)
        hh = jnp.maximum(zacc_ref[i, :n] / den, 0.0)
        mu = jnp.mean(hh, axis=0)
        var = jnp.mean((hh - mu) ** 2, axis=0)
        hh = (hh - mu) * lax.rsqrt(var + 1e-5) * gamma_ref[i] + beta_ref[i]
        outs.append(hh)
    w = w_ref[...]
    t0 = _dot_bf16(outs[0], w) + b_ref[...]
    t1 = _dot_bf16(outs[1], w) + b_ref[...]
    m = jnp.maximum(t0, t1)
    e0 = jnp.exp(t0 - m)
    e1 = jnp.exp(t1 - m)
    inv = 1.0 / (e0 + e1)
    rb = lambda x: x.astype(jnp.bfloat16).astype(jnp.float32)
    hn = rb(e0 * inv) * rb(outs[0]) + rb(e1 * inv) * rb(outs[1])
    hnext_ref[...] = hn
    hmean_ref[...] = jnp.mean(hn, axis=0)


@functools.cache
def _make_edge_kernel(n, e):
    npad = ((n + 8 * NT - 1) // (8 * NT)) * (8 * NT)
    nchunk = e // CHUNK
    rows_per_tile = npad // NT
    outer = (nchunk + NT - 1) // NT
    mesh = plsc.VectorSubcoreMesh(core_axis_name="c", subcore_axis_name="s")

    @functools.partial(
        pl.kernel,
        out_type=(jax.ShapeDtypeStruct((HEADS, npad, H), jnp.float32),
                  jax.ShapeDtypeStruct((HEADS * npad,), jnp.float32)),
        mesh=mesh,
        scratch_types=[
            pltpu.VMEM((2, 1, 128), jnp.int32),
            pltpu.VMEM((2, 1, 128), jnp.int32),
            pltpu.VMEM((2, 1, 128), jnp.int32),
            pltpu.VMEM((2, CHUNK, H), jnp.float32),
            pltpu.VMEM((2, CHUNK), jnp.float32),
            pltpu.VMEM((2, CHUNK), jnp.float32),
            pltpu.VMEM((2, CHUNK), jnp.float32),
            pltpu.VMEM((rows_per_tile,), jnp.float32),
            pltpu.VMEM_SHARED((npad, H), jnp.float32),
            pltpu.VMEM_SHARED((npad,), jnp.float32),
            pltpu.SemaphoreType.DMA,
            pltpu.SemaphoreType.DMA,
            pltpu.SemaphoreType.DMA,
            pltpu.SemaphoreType.DMA,
            pltpu.SemaphoreType.DMA,
            pltpu.SemaphoreType.DMA,
        ],
        compiler_params=pltpu.CompilerParams(needs_layout_passes=False),
    )
    def edge_kernel(src_hbm, dst_hbm, z_hbm, s_hbm, d_hbm,
                    zout_hbm, dout_hbm,
                    zidx, didx, widx, zbuf, sbuf, dbuf, aebuf, dbounce,
                    zacc, dacc, semi0, semi1, semz0, semz1, semsd0, semsd1):
        c = lax.axis_index("c")
        t = lax.axis_index("s")
        cn = c * n
        semi = (semi0, semi1)
        semz = (semz0, semz1)
        semsd = (semsd0, semsd1)

        zero = jnp.zeros((LANES,), jnp.float32)

        def zrow(i, carry):
            for r in range(H // LANES):
                zbuf[0, i, pl.ds(r * LANES, LANES)] = zero
            return carry

        lax.fori_loop(0, CHUNK, zrow, 0)
        for g in range(CHUNK // LANES):
            aebuf[0, pl.ds(g * LANES, LANES)] = zero
        row0 = t * rows_per_tile
        done = 0
        while done < rows_per_tile:
            nrows = min(CHUNK, rows_per_tile - done)
            pltpu.sync_copy(zbuf.at[0, pl.ds(0, nrows)],
                            zacc.at[pl.ds(row0 + done, nrows)])
            pltpu.sync_copy(aebuf.at[0, pl.ds(0, nrows)],
                            dacc.at[pl.ds(row0 + done, nrows)])
            done += nrows
        plsc.subcore_barrier()

        def cid_of(kk):
            return t + NT * kk

        def fire_idx(b, cid):
            @pl.when(cid < nchunk)
            def _():
                pltpu.async_copy(src_hbm.at[pl.ds(cid, 1)], zidx.at[b],
                                 semi[b])
                pltpu.async_copy(dst_hbm.at[pl.ds(cid, 1)], didx.at[b],
                                 semi[b])

        def fire_gathers(b, cid):
            @pl.when(cid < nchunk)
            def _():
                pltpu.make_async_copy(src_hbm.at[pl.ds(cid, 1)], zidx.at[b],
                                      semi[b]).wait()
                pltpu.make_async_copy(dst_hbm.at[pl.ds(cid, 1)], didx.at[b],
                                      semi[b]).wait()
                for g in range(128 // LANES):
                    sl = pl.ds(g * LANES, LANES)
                    zidx[b, 0, sl] = zidx[b, 0, sl] + cn
                    widx[b, 0, sl] = didx[b, 0, sl] + cn
                pltpu.async_copy(z_hbm.at[zidx.at[b, 0]], zbuf.at[b], semz[b])
                pltpu.async_copy(s_hbm.at[zidx.at[b, 0]], sbuf.at[b],
                                 semsd[b])
                pltpu.async_copy(d_hbm.at[widx.at[b, 0]], dbuf.at[b],
                                 semsd[b])

        def work(b, cid):
            @pl.when(cid < nchunk)
            def _():
                pltpu.make_async_copy(s_hbm.at[zidx.at[b, 0]], sbuf.at[b],
                                      semsd[b]).wait()
                pltpu.make_async_copy(d_hbm.at[widx.at[b, 0]], dbuf.at[b],
                                      semsd[b]).wait()
                for g in range(CHUNK // LANES):
                    sl = pl.ds(g * LANES, LANES)
                    ev = sbuf[b, sl] + dbuf[b, sl]
                    ev = jnp.maximum(ev, ev * 0.01)
                    aebuf[b, sl] = jnp.exp(ev)
                pltpu.make_async_copy(z_hbm.at[zidx.at[b, 0]], zbuf.at[b],
                                      semz[b]).wait()

                def scale_body(i, carry):
                    bidx = jnp.zeros((LANES,), jnp.int32) + i
                    w = plsc.load_gather(aebuf.at[b], [bidx])
                    for r in range(H // LANES):
                        sl = pl.ds(r * LANES, LANES)
                        zbuf[b, i, sl] = zbuf[b, i, sl] * w
                    return carry

                lax.fori_loop(0, CHUNK, scale_body, 0, unroll=4)
                pltpu.sync_copy(zbuf.at[b], zacc.at[didx.at[b, 0]], add=True)
                pltpu.sync_copy(aebuf.at[b], dacc.at[didx.at[b, 0]],
                                add=True)

        fire_idx(0, cid_of(0))
        fire_idx(1, cid_of(1))
        fire_gathers(0, cid_of(0))

        def pair_body(kk2, carry):
            kk = 2 * kk2
            for b in (0, 1):
                fire_gathers(1 - b, cid_of(kk + 1))
                work(b, cid_of(kk))
                fire_idx(b, cid_of(kk + 2))
                kk = kk + 1
            return carry

        lax.fori_loop(0, (outer + 1) // 2, pair_body, 0)
        plsc.subcore_barrier()

        pltpu.sync_copy(zacc.at[pl.ds(row0, rows_per_tile)],
                        zout_hbm.at[c, pl.ds(row0, rows_per_tile)])
        pltpu.sync_copy(dacc.at[pl.ds(row0, rows_per_tile)], dbounce)
        pltpu.sync_copy(dbounce,
                        dout_hbm.at[pl.ds(c * npad + row0, rows_per_tile)])

    return edge_kernel


def kernel(x, edge_index, params):
    n, d_in = x.shape
    e = edge_index.shape[1]
    src = edge_index[0].astype(jnp.int32).reshape(e // 128, 128)
    dst = edge_index[1].astype(jnp.int32).reshape(e // 128, 128)
    edge_fn = _make_edge_kernel(n, e)
    npad = ((n + 8 * NT - 1) // (8 * NT)) * (8 * NT)

    tc_prep = pl.pallas_call(
        _tc_prep_body,
        out_shape=(jax.ShapeDtypeStruct((HEADS, n, H), jnp.float32),
                   jax.ShapeDtypeStruct((HEADS, n), jnp.float32),
                   jax.ShapeDtypeStruct((HEADS, n), jnp.float32)),
    )
    tc_combine = pl.pallas_call(
        functools.partial(_tc_combine_body, n=n),
        out_shape=(jax.ShapeDtypeStruct((n, H), jnp.float32),
                   jax.ShapeDtypeStruct((H,), jnp.float32)),
    )

    h = x
    hmean = None
    for j in range(2):
        w_stack = jnp.stack([params['conv{}_{}'.format(j, i)]['W']
                             for i in range(HEADS)])
        asrc = jnp.stack([params['conv{}_{}'.format(j, i)]['a_src']
                          for i in range(HEADS)])
        adst = jnp.stack([params['conv{}_{}'.format(j, i)]['a_dst']
                          for i in range(HEADS)])
        gamma = jnp.stack([params['conv{}_{}'.format(j, i)]['gamma']
                           for i in range(HEADS)])
        beta = jnp.stack([params['conv{}_{}'.format(j, i)]['beta']
                          for i in range(HEADS)])
        z, s, d = tc_prep(h, w_stack, asrc, adst)
        zacc, den = edge_fn(src, dst, z.reshape(HEADS * n, H),
                            s.reshape(HEADS * n), d.reshape(HEADS * n))
        den = den.reshape(HEADS, npad)
        h, hmean = tc_combine(zacc, den, gamma, beta,
                              params['out{}'.format(j)]['w'],
                              params['out{}'.format(j)]['b'])
    return hmean
